# Initial kernel scaffold; baseline (speedup 1.0000x reference)
import functools

import jax
import jax.numpy as jnp
from jax import lax
from jax.experimental import pallas as pl
from jax.experimental.pallas import tpu as pltpu
from jax.experimental.pallas import tpu_sc as plsc

D = 128
CHUNK = 128

_info = plsc.get_sparse_core_info()
NC, NS = _info.num_cores, _info.num_subcores
NW = NC * NS


@jax.jit
def _sc_gather(table, idx2d):
    n_rows, _ = idx2d.shape
    n = n_rows * CHUNK
    b_per_w = n // NW
    n_chunks = b_per_w // CHUNK
    mesh = plsc.VectorSubcoreMesh(core_axis_name="c", subcore_axis_name="s")

    @functools.partial(
        pl.kernel,
        mesh=mesh,
        out_type=jax.ShapeDtypeStruct((n, D), jnp.float32),
        scratch_types=[
            pltpu.VMEM((n_chunks, CHUNK), jnp.int32),
            pltpu.VMEM((CHUNK, D), jnp.float32),
            pltpu.SemaphoreType.DMA,
        ],
    )
    def k(table_hbm, idx_hbm, out_hbm, idx_v, rows_v, sem):
        wid = lax.axis_index("s") * NC + lax.axis_index("c")
        base = wid * b_per_w
        pltpu.sync_copy(idx_hbm.at[pl.ds(wid * n_chunks, n_chunks)], idx_v)

        def body(i, carry):
            pltpu.async_copy(table_hbm.at[idx_v.at[i]], rows_v, sem).wait()
            pltpu.sync_copy(rows_v, out_hbm.at[pl.ds(base + i * CHUNK, CHUNK)])
            return carry

        lax.fori_loop(0, n_chunks, body, 0)

    return k(table, idx2d)


def kernel(inputs, embeddings):
    b, h = inputs.shape
    flat_idx = inputs.reshape(-1).astype(jnp.int32)
    out = _sc_gather(embeddings, flat_idx.reshape(-1, CHUNK))
    return out.reshape(b, h, embeddings.shape[1])


# speedup vs baseline: 2.9770x; 2.9770x over previous
import functools

import jax
import jax.numpy as jnp
from jax import lax
from jax.experimental import pallas as pl
from jax.experimental.pallas import tpu as pltpu
from jax.experimental.pallas import tpu_sc as plsc

D = 128
CHUNK = 128

_info = plsc.get_sparse_core_info()
NC, NS = _info.num_cores, _info.num_subcores
NW = NC * NS


@jax.jit
def _sc_gather(table, idx):
    n = idx.shape[0]
    b_per_w = n // NW
    n_chunks = b_per_w // CHUNK
    mesh = plsc.VectorSubcoreMesh(core_axis_name="c", subcore_axis_name="s")

    @functools.partial(
        pl.kernel,
        mesh=mesh,
        out_type=jax.ShapeDtypeStruct((n, D), jnp.float32),
        scratch_types=[
            pltpu.VMEM((b_per_w,), jnp.int32),
            pltpu.VMEM((CHUNK, D), jnp.float32),
            pltpu.SemaphoreType.DMA,
        ],
    )
    def k(table_hbm, idx_hbm, out_hbm, idx_v, rows_v, sem):
        wid = lax.axis_index("s") * NC + lax.axis_index("c")
        base = wid * b_per_w
        pltpu.sync_copy(idx_hbm.at[pl.ds(base, b_per_w)], idx_v)

        def body(i, carry):
            pltpu.async_copy(
                table_hbm.at[idx_v.at[pl.ds(i * CHUNK, CHUNK)]], rows_v, sem
            ).wait()
            pltpu.sync_copy(rows_v, out_hbm.at[pl.ds(base + i * CHUNK, CHUNK)])
            return carry

        lax.fori_loop(0, n_chunks, body, 0)

    return k(table, idx)


def kernel(inputs, embeddings):
    b, h = inputs.shape
    flat_idx = inputs.reshape(-1).astype(jnp.int32)
    out = _sc_gather(embeddings, flat_idx)
    return out.reshape(b, h, embeddings.shape[1])


# speedup vs baseline: 3.1263x; 1.0501x over previous
import functools

import jax
import jax.numpy as jnp
from jax import lax
from jax.experimental import pallas as pl
from jax.experimental.pallas import tpu as pltpu
from jax.experimental.pallas import tpu_sc as plsc

D = 128
CHUNK = 128

_info = plsc.get_sparse_core_info()
NC, NS = _info.num_cores, _info.num_subcores
NW = NC * NS


@jax.jit
def _sc_gather(table, idx):
    n = idx.shape[0]
    b_per_w = n // NW
    n_chunks = b_per_w // CHUNK
    mesh = plsc.VectorSubcoreMesh(core_axis_name="c", subcore_axis_name="s")

    @functools.partial(
        pl.kernel,
        mesh=mesh,
        out_type=jax.ShapeDtypeStruct((n, D), jnp.float32),
        scratch_types=[
            pltpu.VMEM((b_per_w,), jnp.int32),
            pltpu.VMEM((CHUNK, D), jnp.float32),
            pltpu.VMEM((CHUNK, D), jnp.float32),
            pltpu.SemaphoreType.DMA,
            pltpu.SemaphoreType.DMA,
            pltpu.SemaphoreType.DMA,
            pltpu.SemaphoreType.DMA,
        ],
    )
    def k(table_hbm, idx_hbm, out_hbm, idx_v, buf0, buf1, g0, g1, w0, w1):
        wid = lax.axis_index("s") * NC + lax.axis_index("c")
        base = wid * b_per_w
        pltpu.sync_copy(idx_hbm.at[pl.ds(base, b_per_w)], idx_v)

        def g_start(i, buf, sem):
            pltpu.async_copy(
                table_hbm.at[idx_v.at[pl.ds(i * CHUNK, CHUNK)]], buf, sem
            )

        def g_wait(buf, sem):
            pltpu.make_async_copy(
                table_hbm.at[idx_v.at[pl.ds(0, CHUNK)]], buf, sem
            ).wait()

        def w_start(i, buf, sem):
            pltpu.async_copy(buf, out_hbm.at[pl.ds(base + i * CHUNK, CHUNK)], sem)

        def w_wait(buf, sem):
            pltpu.make_async_copy(buf, out_hbm.at[pl.ds(base, CHUNK)], sem).wait()

        g_start(0, buf0, g0)
        g_wait(buf0, g0)
        g_start(1, buf1, g1)
        w_start(0, buf0, w0)
        g_wait(buf1, g1)
        w_wait(buf0, w0)
        g_start(2, buf0, g0)
        w_start(1, buf1, w1)

        def body(j, carry):
            i = 2 * j
            g_wait(buf0, g0)
            w_wait(buf1, w1)
            g_start(i + 1, buf1, g1)
            w_start(i, buf0, w0)
            g_wait(buf1, g1)
            w_wait(buf0, w0)
            g_start(i + 2, buf0, g0)
            w_start(i + 1, buf1, w1)
            return carry

        lax.fori_loop(1, n_chunks // 2 - 1, body, 0)

        g_wait(buf0, g0)
        w_wait(buf1, w1)
        g_start(n_chunks - 1, buf1, g1)
        w_start(n_chunks - 2, buf0, w0)
        g_wait(buf1, g1)
        w_start(n_chunks - 1, buf1, w1)
        w_wait(buf0, w0)
        w_wait(buf1, w1)

    return k(table, idx)


def kernel(inputs, embeddings):
    b, h = inputs.shape
    flat_idx = inputs.reshape(-1).astype(jnp.int32)
    out = _sc_gather(embeddings, flat_idx)
    return out.reshape(b, h, embeddings.shape[1])


# speedup vs baseline: 3.3542x; 1.0729x over previous
import functools

import jax
import jax.numpy as jnp
from jax import lax
from jax.experimental import pallas as pl
from jax.experimental.pallas import tpu as pltpu
from jax.experimental.pallas import tpu_sc as plsc

D = 128
CHUNK = 128

_info = plsc.get_sparse_core_info()
NC, NS = _info.num_cores, _info.num_subcores
NW = NC * NS


@jax.jit
def _sc_gather(table, idx):
    n = idx.shape[0]
    b_per_w = n // NW
    n_chunks = b_per_w // CHUNK
    mesh = plsc.VectorSubcoreMesh(core_axis_name="c", subcore_axis_name="s")

    NBUF = 5

    @functools.partial(
        pl.kernel,
        mesh=mesh,
        out_type=jax.ShapeDtypeStruct((n, D), jnp.float32),
        scratch_types=[
            pltpu.VMEM((b_per_w,), jnp.int32),
        ]
        + [pltpu.VMEM((CHUNK, D), jnp.float32)] * NBUF
        + [pltpu.SemaphoreType.DMA] * (2 * NBUF),
    )
    def k(table_hbm, idx_hbm, out_hbm, idx_v, *bufs_sems):
        bufs = bufs_sems[:NBUF]
        gsem = bufs_sems[NBUF : 2 * NBUF]
        wsem = bufs_sems[2 * NBUF :]
        wid = lax.axis_index("s") * NC + lax.axis_index("c")
        base = wid * b_per_w
        pltpu.sync_copy(idx_hbm.at[pl.ds(base, b_per_w)], idx_v)

        def g_start(i, b):
            pltpu.async_copy(
                table_hbm.at[idx_v.at[pl.ds(i * CHUNK, CHUNK)]], bufs[b], gsem[b]
            )

        def g_wait(b):
            pltpu.make_async_copy(
                table_hbm.at[idx_v.at[pl.ds(0, CHUNK)]], bufs[b], gsem[b]
            ).wait()

        def w_start(i, b):
            pltpu.async_copy(
                bufs[b], out_hbm.at[pl.ds(base + i * CHUNK, CHUNK)], wsem[b]
            )

        def w_wait(b):
            pltpu.make_async_copy(
                bufs[b], out_hbm.at[pl.ds(base, CHUNK)], wsem[b]
            ).wait()

        for c in range(NBUF):
            g_start(c, c)
        g_wait(0)
        w_start(0, 0)

        def body(j, carry):
            i0 = NBUF * j + 1
            for t in range(NBUF):
                i = i0 + t
                b = (1 + t) % NBUF
                bp = t % NBUF
                g_wait(b)
                w_start(i, b)
                w_wait(bp)
                g_start(i + NBUF - 1, bp)
            return carry

        n_steady = (n_chunks - NBUF) // NBUF
        lax.fori_loop(0, n_steady, body, 0)

        for i in range(n_chunks - NBUF + 1, n_chunks):
            b = i % NBUF
            g_wait(b)
            w_start(i, b)
        for b in range(NBUF):
            w_wait(b)

    return k(table, idx)


def kernel(inputs, embeddings):
    b, h = inputs.shape
    flat_idx = inputs.reshape(-1).astype(jnp.int32)
    out = _sc_gather(embeddings, flat_idx)
    return out.reshape(b, h, embeddings.shape[1])


# speedup vs baseline: 5.9554x; 1.7755x over previous
import functools

import jax
import jax.numpy as jnp
from jax import lax
from jax.experimental import pallas as pl
from jax.experimental.pallas import tpu as pltpu
from jax.experimental.pallas import tpu_sc as plsc

D = 128
NBUF = 8
HP = 56

_info = plsc.get_sparse_core_info()
NC, NS = _info.num_cores, _info.num_subcores
NW = NC * NS


@functools.partial(jax.jit, static_argnums=(2, 3))
def _sc_gather(table, idx_flat, batch, hist):
    b_per_w = batch // NW
    mesh = plsc.VectorSubcoreMesh(core_axis_name="c", subcore_axis_name="s")

    @functools.partial(
        pl.kernel,
        mesh=mesh,
        out_type=jax.ShapeDtypeStruct((batch, hist, D), jnp.float32),
        scratch_types=[
            pltpu.VMEM((b_per_w * HP,), jnp.int32),
        ]
        + [pltpu.VMEM((hist, D), jnp.float32)] * NBUF
        + [pltpu.SemaphoreType.DMA] * (2 * NBUF),
    )
    def k(table_hbm, idx_hbm, out_hbm, idx_v, *bufs_sems):
        bufs = bufs_sems[:NBUF]
        gsem = bufs_sems[NBUF : 2 * NBUF]
        wsem = bufs_sems[2 * NBUF :]
        wid = lax.axis_index("s") * NC + lax.axis_index("c")
        base = wid * b_per_w
        pltpu.sync_copy(idx_hbm.at[pl.ds(base * HP, b_per_w * HP)], idx_v)

        def g_start(i, b):
            pltpu.async_copy(
                table_hbm.at[idx_v.at[pl.ds(i * HP, hist)]], bufs[b], gsem[b]
            )

        def g_wait(b):
            pltpu.make_async_copy(
                table_hbm.at[idx_v.at[pl.ds(0, hist)]], bufs[b], gsem[b]
            ).wait()

        def w_start(i, b):
            pltpu.async_copy(bufs[b], out_hbm.at[base + i], wsem[b])

        def w_wait(b):
            pltpu.make_async_copy(bufs[b], out_hbm.at[base], wsem[b]).wait()

        for i in range(NBUF):
            g_start(i, i)
        g_wait(0)
        w_start(0, 0)

        def body(j, carry):
            i0 = NBUF * j + 1
            for t in range(NBUF):
                i = i0 + t
                b = (1 + t) % NBUF
                bp = t % NBUF
                g_wait(b)
                w_start(i, b)
                w_wait(bp)
                g_start(i + NBUF - 1, bp)
            return carry

        n_steady = (b_per_w - NBUF) // NBUF
        lax.fori_loop(0, n_steady, body, 0)

        for i in range(b_per_w - NBUF + 1, b_per_w):
            b = i % NBUF
            g_wait(b)
            w_start(i, b)
        for b in range(NBUF):
            w_wait(b)

    return k(table, idx_flat)


def kernel(inputs, embeddings):
    batch, hist = inputs.shape
    idx = inputs.astype(jnp.int32)
    idx_pad = jnp.pad(idx, ((0, 0), (0, HP - hist))).reshape(-1)
    return _sc_gather(embeddings, idx_pad, batch, hist)


# speedup vs baseline: 10.7247x; 1.8008x over previous
import functools

import jax
import jax.numpy as jnp
from jax import lax
from jax.experimental import pallas as pl
from jax.experimental.pallas import tpu as pltpu
from jax.experimental.pallas import tpu_sc as plsc

D = 128
CHUNK = 128
NBUF = 5

_info = plsc.get_sparse_core_info()
NC, NS = _info.num_cores, _info.num_subcores
NW = NC * NS


@functools.partial(jax.jit, static_argnums=(2, 3))
def _sc_gather(table, idx_flat, batch, hist):
    n_chunks = hist
    per_w = CHUNK * hist
    mesh = plsc.VectorSubcoreMesh(core_axis_name="c", subcore_axis_name="s")

    @functools.partial(
        pl.kernel,
        mesh=mesh,
        out_type=jax.ShapeDtypeStruct((hist, batch, D), jnp.float32),
        scratch_types=[
            pltpu.VMEM((per_w,), jnp.int32),
        ]
        + [pltpu.VMEM((CHUNK, D), jnp.float32)] * NBUF
        + [pltpu.SemaphoreType.DMA] * (2 * NBUF),
    )
    def k(table_hbm, idx_hbm, out_hbm, idx_v, *bufs_sems):
        bufs = bufs_sems[:NBUF]
        gsem = bufs_sems[NBUF : 2 * NBUF]
        wsem = bufs_sems[2 * NBUF :]
        wid = lax.axis_index("s") * NC + lax.axis_index("c")
        base_b = wid * CHUNK
        pltpu.sync_copy(idx_hbm.at[pl.ds(wid * per_w, per_w)], idx_v)

        def g_start(h, b):
            pltpu.async_copy(
                table_hbm.at[idx_v.at[pl.ds(h * CHUNK, CHUNK)]], bufs[b], gsem[b]
            )

        def g_wait(b):
            pltpu.make_async_copy(
                table_hbm.at[idx_v.at[pl.ds(0, CHUNK)]], bufs[b], gsem[b]
            ).wait()

        def w_start(h, b):
            pltpu.async_copy(
                bufs[b], out_hbm.at[h, pl.ds(base_b, CHUNK)], wsem[b]
            )

        def w_wait(b):
            pltpu.make_async_copy(
                bufs[b], out_hbm.at[0, pl.ds(base_b, CHUNK)], wsem[b]
            ).wait()

        for h in range(NBUF):
            g_start(h, h)
        g_wait(0)
        w_start(0, 0)

        def body(j, carry):
            h0 = NBUF * j + 1
            for t in range(NBUF):
                h = h0 + t
                b = (1 + t) % NBUF
                bp = t % NBUF
                g_wait(b)
                w_start(h, b)
                w_wait(bp)
                g_start(h + NBUF - 1, bp)
            return carry

        n_steady = (n_chunks - NBUF) // NBUF
        lax.fori_loop(0, n_steady, body, 0)

        for h in range(n_chunks - NBUF + 1, n_chunks):
            b = h % NBUF
            g_wait(b)
            w_start(h, b)
        for b in range(NBUF):
            w_wait(b)

    return k(table, idx_flat)


def kernel(inputs, embeddings):
    batch, hist = inputs.shape
    idx = inputs.astype(jnp.int32)
    idx_arr = (
        idx.reshape(NW, CHUNK, hist).transpose(0, 2, 1).reshape(-1)
    )
    out = _sc_gather(embeddings, idx_arr, batch, hist)
    return out.transpose(1, 0, 2)


# speedup vs baseline: 10.7378x; 1.0012x over previous
import functools

import jax
import jax.numpy as jnp
from jax import lax
from jax.experimental import pallas as pl
from jax.experimental.pallas import tpu as pltpu
from jax.experimental.pallas import tpu_sc as plsc

D = 128
CHUNK = 128
NBUF = 7

_info = plsc.get_sparse_core_info()
NC, NS = _info.num_cores, _info.num_subcores
NW = NC * NS


@functools.partial(jax.jit, static_argnums=(2, 3))
def _sc_gather(table, idx_flat, batch, hist):
    n_chunks = hist
    per_w = CHUNK * hist
    mesh = plsc.VectorSubcoreMesh(core_axis_name="c", subcore_axis_name="s")

    @functools.partial(
        pl.kernel,
        mesh=mesh,
        out_type=jax.ShapeDtypeStruct((hist, batch, D), jnp.float32),
        scratch_types=[
            pltpu.VMEM((per_w,), jnp.int32),
        ]
        + [pltpu.VMEM((CHUNK, D), jnp.float32)] * NBUF
        + [pltpu.SemaphoreType.DMA] * (2 * NBUF),
    )
    def k(table_hbm, idx_hbm, out_hbm, idx_v, *bufs_sems):
        bufs = bufs_sems[:NBUF]
        gsem = bufs_sems[NBUF : 2 * NBUF]
        wsem = bufs_sems[2 * NBUF :]
        wid = lax.axis_index("s") * NC + lax.axis_index("c")
        base_b = wid * CHUNK
        pltpu.sync_copy(idx_hbm.at[pl.ds(wid * per_w, per_w)], idx_v)

        def g_start(h, b):
            pltpu.async_copy(
                table_hbm.at[idx_v.at[pl.ds(h * CHUNK, CHUNK)]], bufs[b], gsem[b]
            )

        def g_wait(b):
            pltpu.make_async_copy(
                table_hbm.at[idx_v.at[pl.ds(0, CHUNK)]], bufs[b], gsem[b]
            ).wait()

        def w_start(h, b):
            pltpu.async_copy(
                bufs[b], out_hbm.at[h, pl.ds(base_b, CHUNK)], wsem[b]
            )

        def w_wait(b):
            pltpu.make_async_copy(
                bufs[b], out_hbm.at[0, pl.ds(base_b, CHUNK)], wsem[b]
            ).wait()

        for h in range(NBUF):
            g_start(h, h)
        g_wait(0)
        w_start(0, 0)

        def body(j, carry):
            h0 = NBUF * j + 1
            for t in range(NBUF):
                h = h0 + t
                b = (1 + t) % NBUF
                bp = t % NBUF
                g_wait(b)
                w_start(h, b)
                w_wait(bp)
                g_start(h + NBUF - 1, bp)
            return carry

        n_steady = (n_chunks - NBUF) // NBUF
        lax.fori_loop(0, n_steady, body, 0)

        for h in range(n_steady * NBUF + 1, n_chunks):
            b = h % NBUF
            g_wait(b)
            w_start(h, b)
            if h + NBUF - 1 < n_chunks:
                bp = (h - 1) % NBUF
                w_wait(bp)
                g_start(h + NBUF - 1, bp)
        for h in range(n_chunks - NBUF, n_chunks):
            w_wait(h % NBUF)

    return k(table, idx_flat)


def kernel(inputs, embeddings):
    batch, hist = inputs.shape
    idx = inputs.astype(jnp.int32)
    idx_arr = (
        idx.reshape(NW, CHUNK, hist).transpose(0, 2, 1).reshape(-1)
    )
    out = _sc_gather(embeddings, idx_arr, batch, hist)
    return out.transpose(1, 0, 2)
